# baseline (speedup 1.0000x reference)
import functools

import jax
import jax.numpy as jnp
from jax import lax
from jax.experimental import pallas as pl
from jax.experimental.pallas import tpu as pltpu
from jax.experimental.pallas import tpu_sc as plsc

NC, NS, L = 2, 16, 16
NW = NC * NS
BATCH, HIST, D = 16384, 50, 128
B = BATCH * HIST
CB = 128
BPW = B // NW
NCHUNK = BPW // CB
NBUF = 4
ROUNDS = NCHUNK // NBUF


def _body(inp_hbm, table_hbm, out_hbm, coords_v, idx_v, rows_v, *sems):
    csem = sems[0:NBUF]
    gsem = sems[NBUF:2 * NBUF]
    osem = sems[2 * NBUF:3 * NBUF]
    wid = lax.axis_index("s") * NC + lax.axis_index("c")
    base = wid * BPW

    lane = lax.iota(jnp.int32, L)
    rel = []
    mask = []
    for c in range(3):
        pos = lane * 3 + c
        rel.append([jnp.clip(pos - 16 * k, 0, 15) for k in range(3)])
        mask.append((pos < 16, pos < 32))

    _dnums = lax.GatherDimensionNumbers(
        offset_dims=(), collapsed_slice_dims=(0,), start_index_map=(0,))

    def dyn_gather(v, i):
        return lax.gather(v, i[:, None], dimension_numbers=_dnums,
                          slice_sizes=(1,),
                          mode=lax.GatherScatterMode.PROMISE_IN_BOUNDS)

    def fire_coords(g, b):
        row0 = base + g * CB
        pltpu.async_copy(inp_hbm.at[pl.ds(row0 * 3, CB * 3)], coords_v.at[b],
                         csem[b])

    def wait_coords(b):
        pltpu.make_async_copy(inp_hbm.at[pl.ds(0, CB * 3)], coords_v.at[b],
                              csem[b]).wait()

    def compute_idx(b):
        for j in range(CB // L):
            v = [coords_v[b, pl.ds(j * 3 * L + k * L, L)] for k in range(3)]
            coord = []
            for c in range(3):
                g0, g1, g2 = (dyn_gather(v[k], rel[c][k]) for k in range(3))
                coord.append(jnp.where(mask[c][0], g0,
                                       jnp.where(mask[c][1], g1, g2)))
            idx_v[b, pl.ds(j * L, L)] = (
                (coord[0] * 16 + coord[1]) * 16 + coord[2])

    def fire_gather(b):
        pltpu.async_copy(table_hbm.at[idx_v.at[b]], rows_v.at[b], gsem[b])

    def wait_gather(b):
        pltpu.make_async_copy(table_hbm.at[idx_v.at[b]], rows_v.at[b],
                              gsem[b]).wait()

    def fire_out(g, b):
        pltpu.async_copy(rows_v.at[b], out_hbm.at[pl.ds(base + g * CB, CB)],
                         osem[b])

    def wait_out(b):
        pltpu.make_async_copy(out_hbm.at[pl.ds(base, CB)], rows_v.at[b],
                              osem[b]).wait()

    for b in range(NBUF):
        fire_coords(b, b)

    def round_body(r, carry):
        for b in range(NBUF):
            g = r * NBUF + b
            wait_coords(b)
            compute_idx(b)

            @pl.when(r > 0)
            def _():
                wait_out(b)

            fire_gather(b)
            pb = (b - 1) % NBUF
            if b > 0:
                wait_gather(pb)
                fire_out(g - 1, pb)
            else:
                @pl.when(r > 0)
                def _():
                    wait_gather(pb)
                    fire_out(g - 1, pb)

            @pl.when(r < ROUNDS - 1)
            def _():
                fire_coords(g + NBUF, b)
        return carry

    lax.fori_loop(0, ROUNDS, round_body, 0)

    bl = (NCHUNK - 1) % NBUF
    wait_gather(bl)
    pltpu.sync_copy(rows_v.at[bl], out_hbm.at[pl.ds(base + (NCHUNK - 1) * CB, CB)])
    for b in range(NBUF):
        if b != bl:
            wait_out(b)


_gather = functools.partial(
    pl.kernel,
    out_type=jax.ShapeDtypeStruct((B, D), jnp.float32),
    mesh=plsc.VectorSubcoreMesh(core_axis_name="c", subcore_axis_name="s"),
    scratch_types=(
        [
            pltpu.VMEM((NBUF, CB * 3), jnp.int32),
            pltpu.VMEM((NBUF, CB), jnp.int32),
            pltpu.VMEM((NBUF, CB, D), jnp.float32),
        ]
        + [pltpu.SemaphoreType.DMA] * (3 * NBUF)
    ),
)(_body)


@jax.jit
def kernel(input, table):
    return _gather(input.reshape(B * 3), table).reshape(BATCH, HIST, D)
